# baseline (speedup 1.0000x reference)
import functools

import numpy as np

import jax
import jax.numpy as jnp
from jax import lax
from jax.experimental import pallas as pl
from jax.experimental.pallas import tpu as pltpu
from jax.experimental.pallas import tpu_sc as plsc

K = 15
NEG = 0.1
_rng = np.random.RandomState(42)


def _kp(radius):
    pts = _rng.randn(K, 3)
    pts = pts / np.maximum(np.linalg.norm(pts, axis=1, keepdims=True), 1e-9)
    scales = _rng.rand(K, 1) ** (1.0 / 3.0)
    return np.asarray(pts * scales * radius * 0.66, dtype=np.float32)


R0 = 0.025 * 2.5
KP0, EXT0 = _kp(R0), R0 * 0.6
KP1, EXT1 = _kp(2 * R0), 2 * R0 * 0.6
KP2, EXT2 = _kp(4 * R0), 4 * R0 * 0.6

_NW = 32


def _lrelu(x):
    return jnp.where(x >= 0, x, NEG * x)


@functools.lru_cache(maxsize=None)
def _make_sc_gather(V, D, B, ch):
    b_per_w = B // _NW
    nch = b_per_w // ch
    mesh = plsc.VectorSubcoreMesh(core_axis_name="c", subcore_axis_name="s")

    @functools.partial(
        pl.kernel, mesh=mesh,
        out_type=jax.ShapeDtypeStruct((B, D), jnp.float32),
        compiler_params=pltpu.CompilerParams(use_tc_tiling_on_sc=False),
        scratch_types=[
            pltpu.VMEM((ch,), jnp.int32),
            pltpu.VMEM((ch, D), jnp.float32),
            pltpu.SemaphoreType.DMA,
        ],
    )
    def k(table_hbm, idx_hbm, out_hbm, idx_v, rows_v, sem):
        wid = lax.axis_index("s") * 2 + lax.axis_index("c")
        base = wid * b_per_w

        def body(i, carry):
            off = base + i * ch
            pltpu.sync_copy(idx_hbm.at[pl.ds(off, ch)], idx_v)
            pltpu.async_copy(table_hbm.at[idx_v], rows_v, sem).wait()
            pltpu.sync_copy(rows_v, out_hbm.at[pl.ds(off, ch)])
            return carry

        lax.fori_loop(0, nch, body, 0)

    return k


def _sc_gather(table, idx_flat):
    V, D = table.shape
    B = idx_flat.shape[0]
    ch = min(128, max(8, (96 * 1024) // (D * 4)))
    quant = _NW * ch
    Bp = ((B + quant - 1) // quant) * quant
    if Bp != B:
        idx_flat = jnp.concatenate(
            [idx_flat, jnp.zeros((Bp - B,), jnp.int32)])
    out = _make_sc_gather(V, D, Bp, ch)(table, idx_flat)
    return out[:B]


def _infl(rx, ry, rz, kpt, ext):
    tx = rx[:, :, None] - kpt[0][None, None, :]
    ty = ry[:, :, None] - kpt[1][None, None, :]
    tz = rz[:, :, None] - kpt[2][None, None, :]
    sq = tx * tx + ty * ty + tz * tz
    return jnp.clip(1.0 - jnp.sqrt(sq + 1e-12) * (1.0 / ext), 0.0, None)


_BDN = (((1,), (1,)), ((0,), (0,)))


@functools.lru_cache(maxsize=None)
def _make_layer(Nq, BN, C, Cout, Cin, ext, kp_id, has_sh, strided, next_d):
    def body(*refs):
        it = iter(refs)
        kpt = next(it)[...]
        rx, ry, rz = next(it)[...], next(it)[...], next(it)[...]
        ny = next(it)[...]
        scin = next(it)[...]
        W = next(it)[...]
        u2 = next(it)[...]
        sh = next(it)[...] if has_sh else None
        nu1 = next(it)[...] if next_d else None
        xo_ref = next(it)
        y1_ref = next(it) if next_d else None

        infl = _infl(rx, ry, rz, kpt, ext)
        weighted = lax.dot_general(infl, ny, _BDN,
                                   preferred_element_type=jnp.float32)
        y = weighted[:, 0, :] @ W[0]
        for k in range(1, K):
            y = y + weighted[:, k, :] @ W[k]
        y = _lrelu(y) @ u2
        sc = jnp.max(scin, axis=1) if strided else scin
        if has_sh:
            sc = sc @ sh
        xo = _lrelu(y + sc)
        xo_ref[...] = xo
        if next_d:
            y1_ref[...] = _lrelu(xo @ nu1)

    g = Nq // BN
    r2 = pl.BlockSpec((BN, 32), lambda i: (i, 0))
    in_specs = [pl.BlockSpec((3, K), lambda i: (0, 0)), r2, r2, r2,
                pl.BlockSpec((BN, 32, C), lambda i: (i, 0, 0)),
                (pl.BlockSpec((BN, 32, Cin), lambda i: (i, 0, 0)) if strided
                 else pl.BlockSpec((BN, Cin), lambda i: (i, 0))),
                pl.BlockSpec((K, C, C), lambda i: (0, 0, 0)),
                pl.BlockSpec((C, Cout), lambda i: (0, 0))]
    if has_sh:
        in_specs.append(pl.BlockSpec((Cin, Cout), lambda i: (0, 0)))
    if next_d:
        in_specs.append(pl.BlockSpec((Cout, next_d), lambda i: (0, 0)))
    out_shape = [jax.ShapeDtypeStruct((Nq, Cout), jnp.float32)]
    out_specs = [pl.BlockSpec((BN, Cout), lambda i: (i, 0))]
    if next_d:
        out_shape.append(jax.ShapeDtypeStruct((Nq, next_d), jnp.float32))
        out_specs.append(pl.BlockSpec((BN, next_d), lambda i: (i, 0)))
    return pl.pallas_call(body, grid=(g,), in_specs=in_specs,
                          out_specs=out_specs, out_shape=out_shape)


@functools.lru_cache(maxsize=None)
def _make_b0(Nq, BN, next_d):
    def body(kpt_r, rx_r, ry_r, rz_r, nf_r, w_r, nu1_r, xo_ref, y1_ref):
        infl = _infl(rx_r[...], ry_r[...], rz_r[...], kpt_r[...], EXT0)
        nf = nf_r[...]
        weighted = lax.dot_general(infl, nf, (((1,), (1,)), ((0,), (0,))),
                                   preferred_element_type=jnp.float32)
        xo = _lrelu(weighted @ w_r[...])
        xo_ref[...] = xo
        y1_ref[...] = _lrelu(xo @ nu1_r[...])

    g = Nq // BN
    r2 = pl.BlockSpec((BN, 32), lambda i: (i, 0))
    return pl.pallas_call(
        body, grid=(g,),
        in_specs=[pl.BlockSpec((3, K), lambda i: (0, 0)), r2, r2, r2, r2,
                  pl.BlockSpec((K, 64), lambda i: (0, 0)),
                  pl.BlockSpec((64, next_d), lambda i: (0, 0))],
        out_specs=[pl.BlockSpec((BN, 64), lambda i: (i, 0)),
                   pl.BlockSpec((BN, next_d), lambda i: (i, 0))],
        out_shape=[jax.ShapeDtypeStruct((Nq, 64), jnp.float32),
                   jax.ShapeDtypeStruct((Nq, next_d), jnp.float32)])


def _dec_kernel(xu_ref, skip_ref, du_ref, cw_ref, cb_ref, o_ref):
    cat = jnp.concatenate([xu_ref[...], skip_ref[...]], axis=1)
    y = _lrelu(cat @ du_ref[...])
    o_ref[...] = y @ cw_ref[...] + cb_ref[...][None, :]


def _pad_rows(a, n):
    if a.shape[0] == n:
        return a
    pad = jnp.zeros((n - a.shape[0],) + a.shape[1:], a.dtype)
    return jnp.concatenate([a, pad], axis=0)


def _rels(gp, q_pts, Nq):
    out = []
    for d in range(3):
        out.append(gp[:, d].reshape(Nq, 32) - q_pts[:, d][:, None])
    return out


def kernel(features, points0, points1, points2, b0_W, b1_u1, b1_W, b1_u2, b1_sh, b2_u1, b2_W, b2_u2, b3_u1, b3_W, b3_u2, b3_sh, b4_u1, b4_W, b4_u2, b5_u1, b5_W, b5_u2, b6_u1, b6_W, b6_u2, b6_sh, b7_u1, b7_W, b7_u2, dec_u, coarse_W, coarse_b, neighbors0, neighbors1, neighbors2, pools1, pools2, upsamples1):
    N0p, N1p, N2p, BN = 10240, 2560, 640, 64

    nb0 = _pad_rows(neighbors0.astype(jnp.int32), N0p).reshape(-1)
    nb1 = _pad_rows(neighbors1.astype(jnp.int32), N1p).reshape(-1)
    nb2 = _pad_rows(neighbors2.astype(jnp.int32), N2p).reshape(-1)
    pl1 = _pad_rows(pools1.astype(jnp.int32), N1p).reshape(-1)
    pl2 = _pad_rows(pools2.astype(jnp.int32), N2p).reshape(-1)
    ups = _pad_rows(upsamples1[:, 0].astype(jnp.int32), N1p)

    p0p = _pad_rows(points0, N0p)
    p1p = _pad_rows(points1, N1p)
    p2p = _pad_rows(points2, N2p)
    z5 = jnp.zeros((points0.shape[0], 4), jnp.float32)
    ptsf0 = jnp.concatenate([points0, features, z5], axis=1)
    pts1t = jnp.concatenate(
        [points1, jnp.zeros((points1.shape[0], 5), jnp.float32)], axis=1)
    pts2t = jnp.concatenate(
        [points2, jnp.zeros((points2.shape[0], 5), jnp.float32)], axis=1)

    g0 = _sc_gather(ptsf0, nb0)
    rx0, ry0, rz0 = _rels(g0, p0p, N0p)
    nf0 = g0[:, 3].reshape(N0p, 32)
    kp0t, kp1t, kp2t = [jnp.asarray(k.T) for k in (KP0, KP1, KP2)]
    x0, y1b1 = _make_b0(N0p, BN, 32)(kp0t, rx0, ry0, rz0, nf0, b0_W[:, 0, :], b1_u1)

    ny = _sc_gather(y1b1, nb0).reshape(N0p, 32, 32)
    x1, y1b2 = _make_layer(N0p, BN, 32, 128, 64, EXT0, 0, True, False, 32)(
        kp0t, rx0, ry0, rz0, ny, x0, b1_W, b1_u2, b1_sh, b2_u1)

    gp = _sc_gather(ptsf0, pl1)
    rx, ry, rz = _rels(gp, p1p, N1p)
    ny = _sc_gather(y1b2, pl1).reshape(N1p, 32, 32)
    xg = _sc_gather(x1, pl1).reshape(N1p, 32, 128)
    x2, y1b3 = _make_layer(N1p, BN, 32, 128, 128, EXT0, 0, False, True, 64)(
        kp0t, rx, ry, rz, ny, xg, b2_W, b2_u2, b3_u1)

    gp = _sc_gather(pts1t, nb1)
    rx, ry, rz = _rels(gp, p1p, N1p)
    ny = _sc_gather(y1b3, nb1).reshape(N1p, 32, 64)
    x3, y1b4 = _make_layer(N1p, BN, 64, 256, 128, EXT1, 1, True, False, 64)(
        kp1t, rx, ry, rz, ny, x2, b3_W, b3_u2, b3_sh, b4_u1)
    ny = _sc_gather(y1b4, nb1).reshape(N1p, 32, 64)
    x4, y1b5 = _make_layer(N1p, BN, 64, 256, 256, EXT1, 1, False, False, 64)(
        kp1t, rx, ry, rz, ny, x3, b4_W, b4_u2, b5_u1)
    skip1 = x4

    gp = _sc_gather(pts1t, pl2)
    rx, ry, rz = _rels(gp, p2p, N2p)
    ny = _sc_gather(y1b5, pl2).reshape(N2p, 32, 64)
    xg = _sc_gather(x4, pl2).reshape(N2p, 32, 256)
    x5, y1b6 = _make_layer(N2p, BN, 64, 256, 256, EXT1, 1, False, True, 128)(
        kp1t, rx, ry, rz, ny, xg, b5_W, b5_u2, b6_u1)

    gp = _sc_gather(pts2t, nb2)
    rx, ry, rz = _rels(gp, p2p, N2p)
    ny = _sc_gather(y1b6, nb2).reshape(N2p, 32, 128)
    x6, y1b7 = _make_layer(N2p, BN, 128, 512, 256, EXT2, 2, True, False, 128)(
        kp2t, rx, ry, rz, ny, x5, b6_W, b6_u2, b6_sh, b7_u1)
    ny = _sc_gather(y1b7, nb2).reshape(N2p, 32, 128)
    (x7,) = _make_layer(N2p, BN, 128, 512, 512, EXT2, 2, False, False, 0)(
        kp2t, rx, ry, rz, ny, x6, b7_W, b7_u2)

    xu = _sc_gather(x7, ups)
    out = pl.pallas_call(
        _dec_kernel,
        out_shape=jax.ShapeDtypeStruct((N1p, coarse_W.shape[1]), jnp.float32),
    )(xu, skip1, dec_u, coarse_W, coarse_b)
    return out[:2500]


# speedup vs baseline: 1.0518x; 1.0518x over previous
import functools

import numpy as np

import jax
import jax.numpy as jnp
from jax import lax
from jax.experimental import pallas as pl
from jax.experimental.pallas import tpu as pltpu
from jax.experimental.pallas import tpu_sc as plsc

K = 15
NEG = 0.1
_rng = np.random.RandomState(42)


def _kp(radius):
    pts = _rng.randn(K, 3)
    pts = pts / np.maximum(np.linalg.norm(pts, axis=1, keepdims=True), 1e-9)
    scales = _rng.rand(K, 1) ** (1.0 / 3.0)
    return np.asarray(pts * scales * radius * 0.66, dtype=np.float32)


R0 = 0.025 * 2.5
KP0, EXT0 = _kp(R0), R0 * 0.6
KP1, EXT1 = _kp(2 * R0), 2 * R0 * 0.6
KP2, EXT2 = _kp(4 * R0), 4 * R0 * 0.6

_NW = 32


def _lrelu(x):
    return jnp.where(x >= 0, x, NEG * x)


@functools.lru_cache(maxsize=None)
def _make_sc_gather(tabs, B, IB):
    nt = len(tabs)
    b_per_w = B // _NW
    nib = b_per_w // IB
    sz = min(128, IB)
    ns = IB // sz
    mesh = plsc.VectorSubcoreMesh(core_axis_name="c", subcore_axis_name="s")
    scratch = [pltpu.VMEM((IB,), jnp.int32)]
    scratch += [pltpu.VMEM((IB, D), jnp.float32) for (_, D) in tabs]
    scratch += [pltpu.SemaphoreType.DMA]

    @functools.partial(
        pl.kernel, mesh=mesh,
        out_type=[jax.ShapeDtypeStruct((B, D), jnp.float32) for (_, D) in tabs],
        compiler_params=pltpu.CompilerParams(use_tc_tiling_on_sc=False),
        scratch_types=scratch,
    )
    def k(*refs):
        tables = refs[:nt]
        idx_hbm = refs[nt]
        outs = refs[nt + 1:2 * nt + 1]
        idx_v = refs[2 * nt + 1]
        rows = refs[2 * nt + 2:3 * nt + 2]
        sem = refs[3 * nt + 2]
        wid = lax.axis_index("s") * 2 + lax.axis_index("c")
        base = wid * b_per_w

        def body(i, carry):
            off = base + i * IB
            pltpu.sync_copy(idx_hbm.at[pl.ds(off, IB)], idx_v)
            cps = []
            for t in range(nt):
                for s in range(ns):
                    cps.append(pltpu.async_copy(
                        tables[t].at[idx_v.at[pl.ds(s * sz, sz)]],
                        rows[t].at[pl.ds(s * sz, sz)], sem))
            for cp in cps:
                cp.wait()
            for t in range(nt):
                pltpu.sync_copy(rows[t], outs[t].at[pl.ds(off, IB)])
            return carry

        lax.fori_loop(0, nib, body, 0)

    return k


_VCAP = (360 * 1024) // 4


def _sc_gather_multi(tables, idx_flat):
    B = idx_flat.shape[0]
    sum_d = sum(t.shape[1] for t in tables)
    b_per_w = B // _NW
    assert B % _NW == 0
    cap = max(8, _VCAP // sum_d)
    IB = None
    c = 128
    while c <= min(cap, b_per_w, 2048):
        if b_per_w % c == 0:
            IB = c
        c += 128
    if IB is None:
        IB = b_per_w if (b_per_w <= cap and b_per_w <= 128) else 8
        while b_per_w % IB or IB > min(cap, 128):
            IB -= 8
    tabs = tuple((t.shape[0], t.shape[1]) for t in tables)
    return _make_sc_gather(tabs, B, IB)(*tables, idx_flat)


def _sc_gather(table, idx_flat):
    return _sc_gather_multi((table,), idx_flat)[0]


def _infl(rx, ry, rz, kpt, ext):
    tx = rx[:, :, None] - kpt[0][None, None, :]
    ty = ry[:, :, None] - kpt[1][None, None, :]
    tz = rz[:, :, None] - kpt[2][None, None, :]
    sq = tx * tx + ty * ty + tz * tz
    return jnp.clip(1.0 - jnp.sqrt(sq + 1e-12) * (1.0 / ext), 0.0, None)


_BDN = (((1,), (1,)), ((0,), (0,)))


@functools.lru_cache(maxsize=None)
def _make_layer(Nq, BN, C, Cout, Cin, ext, kp_id, has_sh, strided, next_d):
    def body(*refs):
        it = iter(refs)
        kpt = next(it)[...]
        rx, ry, rz = next(it)[...], next(it)[...], next(it)[...]
        ny = next(it)[...]
        scin = next(it)[...]
        W = next(it)[...]
        u2 = next(it)[...]
        sh = next(it)[...] if has_sh else None
        nu1 = next(it)[...] if next_d else None
        xo_ref = next(it)
        y1_ref = next(it) if next_d else None

        infl = _infl(rx, ry, rz, kpt, ext)
        weighted = lax.dot_general(infl, ny, _BDN,
                                   preferred_element_type=jnp.float32)
        y = weighted[:, 0, :] @ W[0]
        for k in range(1, K):
            y = y + weighted[:, k, :] @ W[k]
        y = _lrelu(y) @ u2
        sc = jnp.max(scin, axis=1) if strided else scin
        if has_sh:
            sc = sc @ sh
        xo = _lrelu(y + sc)
        xo_ref[...] = xo
        if next_d:
            y1_ref[...] = _lrelu(xo @ nu1)

    g = Nq // BN
    r2 = pl.BlockSpec((BN, 32), lambda i: (i, 0))
    in_specs = [pl.BlockSpec((3, K), lambda i: (0, 0)), r2, r2, r2,
                pl.BlockSpec((BN, 32, C), lambda i: (i, 0, 0)),
                (pl.BlockSpec((BN, 32, Cin), lambda i: (i, 0, 0)) if strided
                 else pl.BlockSpec((BN, Cin), lambda i: (i, 0))),
                pl.BlockSpec((K, C, C), lambda i: (0, 0, 0)),
                pl.BlockSpec((C, Cout), lambda i: (0, 0))]
    if has_sh:
        in_specs.append(pl.BlockSpec((Cin, Cout), lambda i: (0, 0)))
    if next_d:
        in_specs.append(pl.BlockSpec((Cout, next_d), lambda i: (0, 0)))
    out_shape = [jax.ShapeDtypeStruct((Nq, Cout), jnp.float32)]
    out_specs = [pl.BlockSpec((BN, Cout), lambda i: (i, 0))]
    if next_d:
        out_shape.append(jax.ShapeDtypeStruct((Nq, next_d), jnp.float32))
        out_specs.append(pl.BlockSpec((BN, next_d), lambda i: (i, 0)))
    return pl.pallas_call(body, grid=(g,), in_specs=in_specs,
                          out_specs=out_specs, out_shape=out_shape)


@functools.lru_cache(maxsize=None)
def _make_b0(Nq, BN, next_d):
    def body(kpt_r, rx_r, ry_r, rz_r, nf_r, w_r, nu1_r, xo_ref, y1_ref):
        infl = _infl(rx_r[...], ry_r[...], rz_r[...], kpt_r[...], EXT0)
        nf = nf_r[...]
        weighted = lax.dot_general(infl, nf, (((1,), (1,)), ((0,), (0,))),
                                   preferred_element_type=jnp.float32)
        xo = _lrelu(weighted @ w_r[...])
        xo_ref[...] = xo
        y1_ref[...] = _lrelu(xo @ nu1_r[...])

    g = Nq // BN
    r2 = pl.BlockSpec((BN, 32), lambda i: (i, 0))
    return pl.pallas_call(
        body, grid=(g,),
        in_specs=[pl.BlockSpec((3, K), lambda i: (0, 0)), r2, r2, r2, r2,
                  pl.BlockSpec((K, 64), lambda i: (0, 0)),
                  pl.BlockSpec((64, next_d), lambda i: (0, 0))],
        out_specs=[pl.BlockSpec((BN, 64), lambda i: (i, 0)),
                   pl.BlockSpec((BN, next_d), lambda i: (i, 0))],
        out_shape=[jax.ShapeDtypeStruct((Nq, 64), jnp.float32),
                   jax.ShapeDtypeStruct((Nq, next_d), jnp.float32)])


def _dec_kernel(xu_ref, skip_ref, du_ref, cw_ref, cb_ref, o_ref):
    cat = jnp.concatenate([xu_ref[...], skip_ref[...]], axis=1)
    y = _lrelu(cat @ du_ref[...])
    o_ref[...] = y @ cw_ref[...] + cb_ref[...][None, :]


def _pad_rows(a, n):
    if a.shape[0] == n:
        return a
    pad = jnp.zeros((n - a.shape[0],) + a.shape[1:], a.dtype)
    return jnp.concatenate([a, pad], axis=0)


def _rels(gp, q_pts, Nq):
    out = []
    for d in range(3):
        out.append(gp[:, d].reshape(Nq, 32) - q_pts[:, d][:, None])
    return out


def kernel(features, points0, points1, points2, b0_W, b1_u1, b1_W, b1_u2, b1_sh, b2_u1, b2_W, b2_u2, b3_u1, b3_W, b3_u2, b3_sh, b4_u1, b4_W, b4_u2, b5_u1, b5_W, b5_u2, b6_u1, b6_W, b6_u2, b6_sh, b7_u1, b7_W, b7_u2, dec_u, coarse_W, coarse_b, neighbors0, neighbors1, neighbors2, pools1, pools2, upsamples1):
    N0p, N1p, N2p, BN = 10240, 2560, 640, 64

    nb0 = _pad_rows(neighbors0.astype(jnp.int32), N0p).reshape(-1)
    nb1 = _pad_rows(neighbors1.astype(jnp.int32), N1p).reshape(-1)
    nb2 = _pad_rows(neighbors2.astype(jnp.int32), N2p).reshape(-1)
    pl1 = _pad_rows(pools1.astype(jnp.int32), N1p).reshape(-1)
    pl2 = _pad_rows(pools2.astype(jnp.int32), N2p).reshape(-1)
    ups = _pad_rows(upsamples1[:, 0].astype(jnp.int32), N1p)

    p0p = _pad_rows(points0, N0p)
    p1p = _pad_rows(points1, N1p)
    p2p = _pad_rows(points2, N2p)
    z5 = jnp.zeros((points0.shape[0], 4), jnp.float32)
    ptsf0 = jnp.concatenate([points0, features, z5], axis=1)
    pts1t = jnp.concatenate(
        [points1, jnp.zeros((points1.shape[0], 5), jnp.float32)], axis=1)
    pts2t = jnp.concatenate(
        [points2, jnp.zeros((points2.shape[0], 5), jnp.float32)], axis=1)

    g0 = _sc_gather(ptsf0, nb0)
    rx0, ry0, rz0 = _rels(g0, p0p, N0p)
    nf0 = g0[:, 3].reshape(N0p, 32)
    kp0t, kp1t, kp2t = [jnp.asarray(k.T) for k in (KP0, KP1, KP2)]
    x0, y1b1 = _make_b0(N0p, BN, 32)(kp0t, rx0, ry0, rz0, nf0, b0_W[:, 0, :], b1_u1)

    ny = _sc_gather(y1b1, nb0).reshape(N0p, 32, 32)
    x1, y1b2 = _make_layer(N0p, BN, 32, 128, 64, EXT0, 0, True, False, 32)(
        kp0t, rx0, ry0, rz0, ny, x0, b1_W, b1_u2, b1_sh, b2_u1)

    gp, ny, xg = _sc_gather_multi((ptsf0, y1b2, x1), pl1)
    rx, ry, rz = _rels(gp, p1p, N1p)
    ny = ny.reshape(N1p, 32, 32)
    xg = xg.reshape(N1p, 32, 128)
    x2, y1b3 = _make_layer(N1p, BN, 32, 128, 128, EXT0, 0, False, True, 64)(
        kp0t, rx, ry, rz, ny, xg, b2_W, b2_u2, b3_u1)

    gp, ny = _sc_gather_multi((pts1t, y1b3), nb1)
    rx, ry, rz = _rels(gp, p1p, N1p)
    ny = ny.reshape(N1p, 32, 64)
    x3, y1b4 = _make_layer(N1p, BN, 64, 256, 128, EXT1, 1, True, False, 64)(
        kp1t, rx, ry, rz, ny, x2, b3_W, b3_u2, b3_sh, b4_u1)
    ny = _sc_gather(y1b4, nb1).reshape(N1p, 32, 64)
    x4, y1b5 = _make_layer(N1p, BN, 64, 256, 256, EXT1, 1, False, False, 64)(
        kp1t, rx, ry, rz, ny, x3, b4_W, b4_u2, b5_u1)
    skip1 = x4

    gp, ny, xg = _sc_gather_multi((pts1t, y1b5, x4), pl2)
    rx, ry, rz = _rels(gp, p2p, N2p)
    ny = ny.reshape(N2p, 32, 64)
    xg = xg.reshape(N2p, 32, 256)
    x5, y1b6 = _make_layer(N2p, BN, 64, 256, 256, EXT1, 1, False, True, 128)(
        kp1t, rx, ry, rz, ny, xg, b5_W, b5_u2, b6_u1)

    gp, ny = _sc_gather_multi((pts2t, y1b6), nb2)
    rx, ry, rz = _rels(gp, p2p, N2p)
    ny = ny.reshape(N2p, 32, 128)
    x6, y1b7 = _make_layer(N2p, BN, 128, 512, 256, EXT2, 2, True, False, 128)(
        kp2t, rx, ry, rz, ny, x5, b6_W, b6_u2, b6_sh, b7_u1)
    ny = _sc_gather(y1b7, nb2).reshape(N2p, 32, 128)
    (x7,) = _make_layer(N2p, BN, 128, 512, 512, EXT2, 2, False, False, 0)(
        kp2t, rx, ry, rz, ny, x6, b7_W, b7_u2)

    xu = _sc_gather(x7, ups)
    out = pl.pallas_call(
        _dec_kernel,
        out_shape=jax.ShapeDtypeStruct((N1p, coarse_W.shape[1]), jnp.float32),
    )(xu, skip1, dec_u, coarse_W, coarse_b)
    return out[:2500]
